# initial kernel scaffold (unmeasured)
import jax
import jax.numpy as jnp
from jax import lax
from jax.experimental import pallas as pl
from jax.experimental.pallas import tpu as pltpu


def kernel(
    t,
):
    def body(*refs):
        pass

    out_shape = jax.ShapeDtypeStruct(..., jnp.float32)
    return pl.pallas_call(body, out_shape=out_shape)(...)


# baseline (device time: 182553 ns/iter reference)
import jax
import jax.numpy as jnp
from jax import lax
from jax.experimental import pallas as pl
from jax.experimental.pallas import tpu as pltpu

N_DEV = 8


def kernel(t):
    m, n = t.shape
    ch = m // N_DEV

    def body(x_ref, out_ref, acc_ref, rs_z, rs_y, rs_x, send_sems, recv_sems):
        my = lax.axis_index("i")
        px = my ^ 1
        py = my ^ 3
        pz = my ^ 4

        bsem = pltpu.get_barrier_semaphore()
        for p in (px, py, pz):
            pl.semaphore_signal(
                bsem, inc=1, device_id=(p,), device_id_type=pl.DeviceIdType.MESH
            )
        pl.semaphore_wait(bsem, 3)

        acc_ref[...] = x_ref[...].astype(jnp.bfloat16)

        zbit = my >> 2
        ybit = (my >> 1) & 1

        def xchg(idx, src, dst, partner):
            rdma = pltpu.make_async_remote_copy(
                src_ref=src,
                dst_ref=dst,
                send_sem=send_sems.at[idx],
                recv_sem=recv_sems.at[idx],
                device_id=(partner,),
                device_id_type=pl.DeviceIdType.MESH,
            )
            rdma.start()
            rdma.wait()

        def accum(start_rows, nrows, buf):
            a = acc_ref[pl.ds(start_rows, nrows), :].astype(jnp.float32)
            b = buf[...].astype(jnp.float32)
            acc_ref[pl.ds(start_rows, nrows), :] = (a + b).astype(jnp.bfloat16)

        keep4 = zbit * (4 * ch)
        send4 = (zbit ^ 1) * (4 * ch)
        xchg(0, acc_ref.at[pl.ds(send4, 4 * ch)], rs_z, pz)
        accum(keep4, 4 * ch, rs_z)

        keep2 = keep4 + ybit * (2 * ch)
        send2 = keep4 + (ybit ^ 1) * (2 * ch)
        xchg(1, acc_ref.at[pl.ds(send2, 2 * ch)], rs_y, py)
        accum(keep2, 2 * ch, rs_y)

        keep1 = my * ch
        send1 = (my ^ 1) * ch
        xchg(2, acc_ref.at[pl.ds(send1, ch)], rs_x, px)
        accum(keep1, ch, rs_x)

        s = acc_ref[pl.ds(keep1, ch), :].astype(jnp.float32)
        r = jnp.maximum(s, 0.0)
        f = jnp.tanh(s) * s * s + r * r * r
        out_ref[pl.ds(keep1, ch), :] = f.astype(jnp.bfloat16)

        xchg(3, out_ref.at[pl.ds(keep1, ch)], out_ref.at[pl.ds(keep1, ch)], px)
        g2 = (my & ~1) * ch
        xchg(4, out_ref.at[pl.ds(g2, 2 * ch)], out_ref.at[pl.ds(g2, 2 * ch)], py)
        g4 = (my & ~3) * ch
        xchg(5, out_ref.at[pl.ds(g4, 4 * ch)], out_ref.at[pl.ds(g4, 4 * ch)], pz)

    return pl.pallas_call(
        body,
        out_shape=jax.ShapeDtypeStruct((m, n), jnp.bfloat16),
        in_specs=[pl.BlockSpec(memory_space=pltpu.VMEM)],
        out_specs=pl.BlockSpec(memory_space=pltpu.VMEM),
        scratch_shapes=[
            pltpu.VMEM((m, n), jnp.bfloat16),
            pltpu.VMEM((m // 2, n), jnp.bfloat16),
            pltpu.VMEM((m // 4, n), jnp.bfloat16),
            pltpu.VMEM((m // 8, n), jnp.bfloat16),
            pltpu.SemaphoreType.DMA((6,)),
            pltpu.SemaphoreType.DMA((6,)),
        ],
        compiler_params=pltpu.CompilerParams(collective_id=0),
    )(t)


# device time: 78580 ns/iter; 2.3231x vs baseline; 2.3231x over previous
import jax
import jax.numpy as jnp
from jax import lax
from jax.experimental import pallas as pl
from jax.experimental.pallas import tpu as pltpu

N_DEV = 8

STREAMS = (
    (0, 1408, ("z", "y", "x")),
    (1408, 1408, ("y", "x", "z")),
    (2816, 1280, ("x", "z", "y")),
)


def kernel(t):
    m, n = t.shape

    def body(x_ref, out_ref, *rest):
        acc_ref = rest[0]
        rs_bufs = [rest[1 + 3 * s : 4 + 3 * s] for s in range(len(STREAMS))]
        send_sems, recv_sems = rest[-2], rest[-1]

        my = lax.axis_index("i")
        partner = {"x": my ^ 1, "y": my ^ 3, "z": my ^ 4}
        bit = {"x": (my ^ (my >> 1)) & 1, "y": (my >> 1) & 1, "z": my >> 2}

        bsem = pltpu.get_barrier_semaphore()
        for ax in ("x", "y", "z"):
            pl.semaphore_signal(
                bsem,
                inc=1,
                device_id=(partner[ax],),
                device_id_type=pl.DeviceIdType.MESH,
            )
        pl.semaphore_wait(bsem, 3)

        geo = []
        for base, rows, order in STREAMS:
            ch = rows // 8
            b2, b1, b0 = bit[order[0]], bit[order[1]], bit[order[2]]
            geo.append(
                dict(
                    base=base,
                    ch=ch,
                    order=order,
                    b2=b2,
                    b1=b1,
                    b0=b0,
                    c_own=b2 * 4 + b1 * 2 + b0,
                )
            )

        rdmas = {}

        def xchg(s, stage, src, dst, ax):
            idx = s * 6 + stage
            r = pltpu.make_async_remote_copy(
                src_ref=src,
                dst_ref=dst,
                send_sem=send_sems.at[idx],
                recv_sem=recv_sems.at[idx],
                device_id=(partner[ax],),
                device_id_type=pl.DeviceIdType.MESH,
            )
            r.start()
            rdmas[(s, stage)] = r

        def accum(start_rows, nrows, buf):
            a = acc_ref[pl.ds(start_rows, nrows), :].astype(jnp.float32)
            b = buf[...].astype(jnp.float32)
            acc_ref[pl.ds(start_rows, nrows), :] = (a + b).astype(jnp.bfloat16)

        for s, g in enumerate(geo):
            send4 = g["base"] + (g["b2"] ^ 1) * 4 * g["ch"]
            acc_ref[pl.ds(send4, 4 * g["ch"]), :] = x_ref[
                pl.ds(send4, 4 * g["ch"]), :
            ].astype(jnp.bfloat16)
            xchg(s, 0, acc_ref.at[pl.ds(send4, 4 * g["ch"])], rs_bufs[s][0],
                 g["order"][0])
        for s, g in enumerate(geo):
            keep4 = g["base"] + g["b2"] * 4 * g["ch"]
            acc_ref[pl.ds(keep4, 4 * g["ch"]), :] = x_ref[
                pl.ds(keep4, 4 * g["ch"]), :
            ].astype(jnp.bfloat16)
            g["keep4"] = keep4

        for s, g in enumerate(geo):
            rdmas[(s, 0)].wait()
            accum(g["keep4"], 4 * g["ch"], rs_bufs[s][0])
            send2 = g["keep4"] + (g["b1"] ^ 1) * 2 * g["ch"]
            xchg(s, 1, acc_ref.at[pl.ds(send2, 2 * g["ch"])], rs_bufs[s][1],
                 g["order"][1])
            g["keep2"] = g["keep4"] + g["b1"] * 2 * g["ch"]

        for s, g in enumerate(geo):
            rdmas[(s, 1)].wait()
            accum(g["keep2"], 2 * g["ch"], rs_bufs[s][1])
            send1 = g["keep2"] + (g["b0"] ^ 1) * g["ch"]
            xchg(s, 2, acc_ref.at[pl.ds(send1, g["ch"])], rs_bufs[s][2],
                 g["order"][2])
            g["own"] = g["base"] + g["c_own"] * g["ch"]

        for s, g in enumerate(geo):
            rdmas[(s, 2)].wait()
            accum(g["own"], g["ch"], rs_bufs[s][2])
            sv = acc_ref[pl.ds(g["own"], g["ch"]), :].astype(jnp.float32)
            rv = jnp.maximum(sv, 0.0)
            fv = jnp.tanh(sv) * sv * sv + rv * rv * rv
            out_ref[pl.ds(g["own"], g["ch"]), :] = fv.astype(jnp.bfloat16)
            src = out_ref.at[pl.ds(g["own"], g["ch"])]
            xchg(s, 3, src, src, g["order"][2])

        for s, g in enumerate(geo):
            rdmas[(s, 3)].wait()
            g2 = g["base"] + (g["c_own"] & ~1) * g["ch"]
            src = out_ref.at[pl.ds(g2, 2 * g["ch"])]
            xchg(s, 4, src, src, g["order"][1])
        for s, g in enumerate(geo):
            rdmas[(s, 4)].wait()
            g4 = g["base"] + (g["c_own"] & ~3) * g["ch"]
            src = out_ref.at[pl.ds(g4, 4 * g["ch"])]
            xchg(s, 5, src, src, g["order"][0])
        for s in range(len(geo)):
            rdmas[(s, 5)].wait()

    scratch = [pltpu.VMEM((m, n), jnp.bfloat16)]
    for _, rows, _ in STREAMS:
        ch = rows // 8
        scratch += [
            pltpu.VMEM((4 * ch, n), jnp.bfloat16),
            pltpu.VMEM((2 * ch, n), jnp.bfloat16),
            pltpu.VMEM((ch, n), jnp.bfloat16),
        ]
    n_x = 6 * len(STREAMS)
    scratch += [pltpu.SemaphoreType.DMA((n_x,)), pltpu.SemaphoreType.DMA((n_x,))]

    return pl.pallas_call(
        body,
        out_shape=jax.ShapeDtypeStruct((m, n), jnp.bfloat16),
        in_specs=[pl.BlockSpec(memory_space=pltpu.VMEM)],
        out_specs=pl.BlockSpec(memory_space=pltpu.VMEM),
        scratch_shapes=scratch,
        compiler_params=pltpu.CompilerParams(collective_id=0),
    )(t)


# device time: 78373 ns/iter; 2.3293x vs baseline; 1.0026x over previous
import jax
import jax.numpy as jnp
from jax import lax
from jax.experimental import pallas as pl
from jax.experimental.pallas import tpu as pltpu

N_DEV = 8

STREAMS = (
    (0, 1408, ("z", "y", "x")),
    (1408, 1408, ("y", "x", "z")),
    (2816, 1280, ("x", "z", "y")),
)


def kernel(t):
    m, n = t.shape

    def body(x_ref, out_ref, *rest):
        acc_ref = rest[0]
        rs_bufs = [rest[1 + 3 * s : 4 + 3 * s] for s in range(len(STREAMS))]
        send_sems, recv_sems = rest[-2], rest[-1]

        my = lax.axis_index("i")
        partner = {"x": my ^ 1, "y": my ^ 3, "z": my ^ 4}
        bit = {"x": (my ^ (my >> 1)) & 1, "y": (my >> 1) & 1, "z": my >> 2}

        bsem = pltpu.get_barrier_semaphore()
        for ax in ("x", "y", "z"):
            pl.semaphore_signal(
                bsem,
                inc=1,
                device_id=(partner[ax],),
                device_id_type=pl.DeviceIdType.MESH,
            )
        pl.semaphore_wait(bsem, 3)

        geo = []
        for base, rows, order in STREAMS:
            ch = rows // 8
            b2, b1, b0 = bit[order[0]], bit[order[1]], bit[order[2]]
            geo.append(
                dict(
                    base=base,
                    ch=ch,
                    order=order,
                    b2=b2,
                    b1=b1,
                    b0=b0,
                    c_own=b2 * 4 + b1 * 2 + b0,
                )
            )

        rdmas = {}

        def xchg(s, stage, src, dst, ax):
            idx = s * 6 + stage
            r = pltpu.make_async_remote_copy(
                src_ref=src,
                dst_ref=dst,
                send_sem=send_sems.at[idx],
                recv_sem=recv_sems.at[idx],
                device_id=(partner[ax],),
                device_id_type=pl.DeviceIdType.MESH,
            )
            r.start()
            rdmas[(s, stage)] = r

        def accum(start_rows, nrows, buf):
            a = acc_ref[pl.ds(start_rows, nrows), :].astype(jnp.float32)
            b = buf[...].astype(jnp.float32)
            acc_ref[pl.ds(start_rows, nrows), :] = (a + b).astype(jnp.bfloat16)

        for s, g in enumerate(geo):
            send4 = g["base"] + (g["b2"] ^ 1) * 4 * g["ch"]
            acc_ref[pl.ds(send4, 4 * g["ch"]), :] = x_ref[
                pl.ds(send4, 4 * g["ch"]), :
            ].astype(jnp.bfloat16)
            xchg(s, 0, acc_ref.at[pl.ds(send4, 4 * g["ch"])], rs_bufs[s][0],
                 g["order"][0])
            g["keep4"] = g["base"] + g["b2"] * 4 * g["ch"]

        for s, g in enumerate(geo):
            rdmas[(s, 0)].wait()
            acc_ref[pl.ds(g["keep4"], 4 * g["ch"]), :] = (
                x_ref[pl.ds(g["keep4"], 4 * g["ch"]), :]
                + rs_bufs[s][0][...].astype(jnp.float32)
            ).astype(jnp.bfloat16)
            send2 = g["keep4"] + (g["b1"] ^ 1) * 2 * g["ch"]
            xchg(s, 1, acc_ref.at[pl.ds(send2, 2 * g["ch"])], rs_bufs[s][1],
                 g["order"][1])
            g["keep2"] = g["keep4"] + g["b1"] * 2 * g["ch"]

        for s, g in enumerate(geo):
            rdmas[(s, 1)].wait()
            accum(g["keep2"], 2 * g["ch"], rs_bufs[s][1])
            send1 = g["keep2"] + (g["b0"] ^ 1) * g["ch"]
            xchg(s, 2, acc_ref.at[pl.ds(send1, g["ch"])], rs_bufs[s][2],
                 g["order"][2])
            g["own"] = g["base"] + g["c_own"] * g["ch"]

        for s, g in enumerate(geo):
            rdmas[(s, 2)].wait()
            accum(g["own"], g["ch"], rs_bufs[s][2])
            sv = acc_ref[pl.ds(g["own"], g["ch"]), :].astype(jnp.float32)
            rv = jnp.maximum(sv, 0.0)
            fv = jnp.tanh(sv) * sv * sv + rv * rv * rv
            out_ref[pl.ds(g["own"], g["ch"]), :] = fv.astype(jnp.bfloat16)
            src = out_ref.at[pl.ds(g["own"], g["ch"])]
            xchg(s, 3, src, src, g["order"][2])

        for s, g in enumerate(geo):
            rdmas[(s, 3)].wait()
            g2 = g["base"] + (g["c_own"] & ~1) * g["ch"]
            src = out_ref.at[pl.ds(g2, 2 * g["ch"])]
            xchg(s, 4, src, src, g["order"][1])
        for s, g in enumerate(geo):
            rdmas[(s, 4)].wait()
            g4 = g["base"] + (g["c_own"] & ~3) * g["ch"]
            src = out_ref.at[pl.ds(g4, 4 * g["ch"])]
            xchg(s, 5, src, src, g["order"][0])
        for s in range(len(geo)):
            rdmas[(s, 5)].wait()

    scratch = [pltpu.VMEM((m, n), jnp.bfloat16)]
    for _, rows, _ in STREAMS:
        ch = rows // 8
        scratch += [
            pltpu.VMEM((4 * ch, n), jnp.bfloat16),
            pltpu.VMEM((2 * ch, n), jnp.bfloat16),
            pltpu.VMEM((ch, n), jnp.bfloat16),
        ]
    n_x = 6 * len(STREAMS)
    scratch += [pltpu.SemaphoreType.DMA((n_x,)), pltpu.SemaphoreType.DMA((n_x,))]

    return pl.pallas_call(
        body,
        out_shape=jax.ShapeDtypeStruct((m, n), jnp.bfloat16),
        in_specs=[pl.BlockSpec(memory_space=pltpu.VMEM)],
        out_specs=pl.BlockSpec(memory_space=pltpu.VMEM),
        scratch_shapes=scratch,
        compiler_params=pltpu.CompilerParams(collective_id=0),
    )(t)


# device time: 70322 ns/iter; 2.5960x vs baseline; 1.1145x over previous
import jax
import jax.numpy as jnp
from jax import lax
from jax.experimental import pallas as pl
from jax.experimental.pallas import tpu as pltpu

N_DEV = 8

STREAMS = (
    (0, 1408, ("z", "y", "x")),
    (1408, 1408, ("y", "x", "z")),
    (2816, 1280, ("x", "z", "y")),
)
X_PER_STREAM = 10


def kernel(t):
    m, n = t.shape

    def body(x_ref, out_ref, *rest):
        acc_ref = rest[0]
        rs_bufs = [rest[1 + 3 * s : 4 + 3 * s] for s in range(len(STREAMS))]
        send_sems, recv_sems = rest[-2], rest[-1]

        my = lax.axis_index("i")
        partner = {"x": my ^ 1, "y": my ^ 3, "z": my ^ 4}
        bit = {"x": (my ^ (my >> 1)) & 1, "y": (my >> 1) & 1, "z": my >> 2}

        bsem = pltpu.get_barrier_semaphore()
        for ax in ("x", "y", "z"):
            pl.semaphore_signal(
                bsem,
                inc=1,
                device_id=(partner[ax],),
                device_id_type=pl.DeviceIdType.MESH,
            )
        pl.semaphore_wait(bsem, 3)

        geo = []
        for base, rows, order in STREAMS:
            ch = rows // 8
            b2, b1, b0 = bit[order[0]], bit[order[1]], bit[order[2]]
            g = dict(base=base, ch=ch, order=order, b2=b2, b1=b1, b0=b0)
            g["c_own"] = b2 * 4 + b1 * 2 + b0
            g["keep4"] = base + b2 * 4 * ch
            g["send4"] = base + (b2 ^ 1) * 4 * ch
            g["keep2"] = g["keep4"] + b1 * 2 * ch
            g["send2"] = g["keep4"] + (b1 ^ 1) * 2 * ch
            g["own"] = base + g["c_own"] * ch
            g["send1"] = g["keep2"] + (b0 ^ 1) * ch
            geo.append(g)

        rdmas = {}

        def xchg(s, k, src, dst, ax):
            idx = s * X_PER_STREAM + k
            r = pltpu.make_async_remote_copy(
                src_ref=src,
                dst_ref=dst,
                send_sem=send_sems.at[idx],
                recv_sem=recv_sems.at[idx],
                device_id=(partner[ax],),
                device_id_type=pl.DeviceIdType.MESH,
            )
            r.start()
            rdmas[(s, k)] = r

        for s, g in enumerate(geo):
            ch, off = g["ch"], (g["b1"] ^ 1) * 2 * g["ch"]
            rows_ = pl.ds(g["send4"] + off, 2 * ch)
            acc_ref[rows_, :] = x_ref[rows_, :].astype(jnp.bfloat16)
            xchg(s, 0, acc_ref.at[rows_], rs_bufs[s][0].at[pl.ds(off, 2 * ch)],
                 g["order"][0])
        for s, g in enumerate(geo):
            ch, off = g["ch"], g["b1"] * 2 * g["ch"]
            rows_ = pl.ds(g["send4"] + off, 2 * ch)
            acc_ref[rows_, :] = x_ref[rows_, :].astype(jnp.bfloat16)
            xchg(s, 1, acc_ref.at[rows_], rs_bufs[s][0].at[pl.ds(off, 2 * ch)],
                 g["order"][0])

        for s, g in enumerate(geo):
            ch = g["ch"]
            rdmas[(s, 0)].wait()
            off = (g["b1"] ^ 1) * 2 * ch
            acc_ref[pl.ds(g["send2"], 2 * ch), :] = (
                x_ref[pl.ds(g["send2"], 2 * ch), :]
                + rs_bufs[s][0][pl.ds(off, 2 * ch), :].astype(jnp.float32)
            ).astype(jnp.bfloat16)
            for k, cb in ((2, g["b0"] ^ 1), (3, g["b0"])):
                rows_ = pl.ds(g["send2"] + cb * ch, ch)
                xchg(s, k, acc_ref.at[rows_],
                     rs_bufs[s][1].at[pl.ds(cb * ch, ch)], g["order"][1])

        for s, g in enumerate(geo):
            ch = g["ch"]
            rdmas[(s, 1)].wait()
            off = g["b1"] * 2 * ch
            acc_ref[pl.ds(g["keep2"], 2 * ch), :] = (
                x_ref[pl.ds(g["keep2"], 2 * ch), :]
                + rs_bufs[s][0][pl.ds(off, 2 * ch), :].astype(jnp.float32)
            ).astype(jnp.bfloat16)

        for s, g in enumerate(geo):
            ch = g["ch"]
            rdmas[(s, 2)].wait()
            off = (g["b0"] ^ 1) * ch
            a = acc_ref[pl.ds(g["send1"], ch), :].astype(jnp.float32)
            b = rs_bufs[s][1][pl.ds(off, ch), :].astype(jnp.float32)
            acc_ref[pl.ds(g["send1"], ch), :] = (a + b).astype(jnp.bfloat16)
            xchg(s, 4, acc_ref.at[pl.ds(g["send1"], ch)], rs_bufs[s][2],
                 g["order"][2])

        for s, g in enumerate(geo):
            ch = g["ch"]
            rdmas[(s, 3)].wait()
            off = g["b0"] * ch
            a = acc_ref[pl.ds(g["own"], ch), :].astype(jnp.float32)
            b = rs_bufs[s][1][pl.ds(off, ch), :].astype(jnp.float32)
            acc_ref[pl.ds(g["own"], ch), :] = (a + b).astype(jnp.bfloat16)

        for s, g in enumerate(geo):
            ch = g["ch"]
            rdmas[(s, 4)].wait()
            sv = (
                acc_ref[pl.ds(g["own"], ch), :].astype(jnp.float32)
                + rs_bufs[s][2][...].astype(jnp.float32)
            )
            rv = jnp.maximum(sv, 0.0)
            fv = jnp.tanh(sv) * sv * sv + rv * rv * rv
            out_ref[pl.ds(g["own"], ch), :] = fv.astype(jnp.bfloat16)
            src = out_ref.at[pl.ds(g["own"], ch)]
            xchg(s, 5, src, src, g["order"][2])
            xchg(s, 6, src, src, g["order"][1])

        for s, g in enumerate(geo):
            ch = g["ch"]
            rdmas[(s, 5)].wait()
            src = out_ref.at[pl.ds(g["base"] + (g["c_own"] ^ 1) * ch, ch)]
            xchg(s, 7, src, src, g["order"][1])
            pair = out_ref.at[pl.ds(g["base"] + (g["c_own"] & ~1) * ch, 2 * ch)]
            xchg(s, 8, pair, pair, g["order"][0])

        for s, g in enumerate(geo):
            ch = g["ch"]
            rdmas[(s, 6)].wait()
            rdmas[(s, 7)].wait()
            opair = g["base"] + ((g["c_own"] >> 1) ^ 1) * 2 * ch
            src = out_ref.at[pl.ds(opair, 2 * ch)]
            xchg(s, 9, src, src, g["order"][0])

        for s in range(len(geo)):
            rdmas[(s, 8)].wait()
            rdmas[(s, 9)].wait()

    scratch = [pltpu.VMEM((m, n), jnp.bfloat16)]
    for _, rows, _ in STREAMS:
        ch = rows // 8
        scratch += [
            pltpu.VMEM((4 * ch, n), jnp.bfloat16),
            pltpu.VMEM((2 * ch, n), jnp.bfloat16),
            pltpu.VMEM((ch, n), jnp.bfloat16),
        ]
    n_x = X_PER_STREAM * len(STREAMS)
    scratch += [pltpu.SemaphoreType.DMA((n_x,)), pltpu.SemaphoreType.DMA((n_x,))]

    return pl.pallas_call(
        body,
        out_shape=jax.ShapeDtypeStruct((m, n), jnp.bfloat16),
        in_specs=[pl.BlockSpec(memory_space=pltpu.VMEM)],
        out_specs=pl.BlockSpec(memory_space=pltpu.VMEM),
        scratch_shapes=scratch,
        compiler_params=pltpu.CompilerParams(collective_id=0),
    )(t)


# device time: 69729 ns/iter; 2.6180x vs baseline; 1.0085x over previous
import jax
import jax.numpy as jnp
from jax import lax
from jax.experimental import pallas as pl
from jax.experimental.pallas import tpu as pltpu

N_DEV = 8

STREAMS = (
    (0, 1408, ("z", "y", "x")),
    (1408, 1408, ("y", "x", "z")),
    (2816, 1280, ("x", "z", "y")),
)
X_PER_STREAM = 13


def kernel(t):
    m, n = t.shape

    def body(x_ref, out_ref, *rest):
        acc_ref = rest[0]
        rs_bufs = [rest[1 + 3 * s : 4 + 3 * s] for s in range(len(STREAMS))]
        send_sems, recv_sems = rest[-2], rest[-1]

        my = lax.axis_index("i")
        partner = {"x": my ^ 1, "y": my ^ 3, "z": my ^ 4}
        bit = {"x": (my ^ (my >> 1)) & 1, "y": (my >> 1) & 1, "z": my >> 2}

        bsem = pltpu.get_barrier_semaphore()
        for ax in ("x", "y", "z"):
            pl.semaphore_signal(
                bsem,
                inc=1,
                device_id=(partner[ax],),
                device_id_type=pl.DeviceIdType.MESH,
            )
        pl.semaphore_wait(bsem, 3)

        geo = []
        for base, rows, order in STREAMS:
            ch = rows // 8
            b2, b1, b0 = bit[order[0]], bit[order[1]], bit[order[2]]
            g = dict(base=base, ch=ch, order=order, b2=b2, b1=b1, b0=b0)
            g["c_own"] = b2 * 4 + b1 * 2 + b0
            g["keep4"] = base + b2 * 4 * ch
            g["send4"] = base + (b2 ^ 1) * 4 * ch
            g["keep2"] = g["keep4"] + b1 * 2 * ch
            g["send2"] = g["keep4"] + (b1 ^ 1) * 2 * ch
            g["own"] = base + g["c_own"] * ch
            g["send1"] = g["keep2"] + (b0 ^ 1) * ch
            geo.append(g)

        rdmas = {}

        def xchg(s, k, src, dst, ax):
            idx = s * X_PER_STREAM + k
            r = pltpu.make_async_remote_copy(
                src_ref=src,
                dst_ref=dst,
                send_sem=send_sems.at[idx],
                recv_sem=recv_sems.at[idx],
                device_id=(partner[ax],),
                device_id_type=pl.DeviceIdType.MESH,
            )
            r.start()
            rdmas[(s, k)] = r

        def chunk_src(s, g, c):
            return out_ref.at[pl.ds(g["base"] + c * g["ch"], g["ch"])]

        for s, g in enumerate(geo):
            ch, off = g["ch"], (g["b1"] ^ 1) * 2 * g["ch"]
            rows_ = pl.ds(g["send4"] + off, 2 * ch)
            acc_ref[rows_, :] = x_ref[rows_, :].astype(jnp.bfloat16)
            xchg(s, 0, acc_ref.at[rows_], rs_bufs[s][0].at[pl.ds(off, 2 * ch)],
                 g["order"][0])

        for s, g in enumerate(geo):
            ch, off = g["ch"], g["b1"] * 2 * g["ch"]
            rows_ = pl.ds(g["send4"] + off, 2 * ch)
            acc_ref[rows_, :] = x_ref[rows_, :].astype(jnp.bfloat16)
            for k, cb in ((1, g["b0"] ^ 1), (2, g["b0"])):
                po = off + cb * ch
                xchg(s, k, acc_ref.at[pl.ds(g["send4"] + po, ch)],
                     rs_bufs[s][0].at[pl.ds(po, ch)], g["order"][0])

        for s, g in enumerate(geo):
            ch = g["ch"]
            rdmas[(s, 0)].wait()
            off = (g["b1"] ^ 1) * 2 * ch
            acc_ref[pl.ds(g["send2"], 2 * ch), :] = (
                x_ref[pl.ds(g["send2"], 2 * ch), :]
                + rs_bufs[s][0][pl.ds(off, 2 * ch), :].astype(jnp.float32)
            ).astype(jnp.bfloat16)
            for k, cb in ((3, g["b0"] ^ 1), (4, g["b0"])):
                rows_ = pl.ds(g["send2"] + cb * ch, ch)
                xchg(s, k, acc_ref.at[rows_],
                     rs_bufs[s][1].at[pl.ds(cb * ch, ch)], g["order"][1])

        for s, g in enumerate(geo):
            ch = g["ch"]
            rdmas[(s, 1)].wait()
            off = g["b1"] * 2 * ch + (g["b0"] ^ 1) * ch
            acc_ref[pl.ds(g["send1"], ch), :] = (
                x_ref[pl.ds(g["send1"], ch), :]
                + rs_bufs[s][0][pl.ds(off, ch), :].astype(jnp.float32)
            ).astype(jnp.bfloat16)

        for s, g in enumerate(geo):
            ch = g["ch"]
            rdmas[(s, 3)].wait()
            off = (g["b0"] ^ 1) * ch
            a = acc_ref[pl.ds(g["send1"], ch), :].astype(jnp.float32)
            b = rs_bufs[s][1][pl.ds(off, ch), :].astype(jnp.float32)
            acc_ref[pl.ds(g["send1"], ch), :] = (a + b).astype(jnp.bfloat16)
            xchg(s, 5, acc_ref.at[pl.ds(g["send1"], ch)], rs_bufs[s][2],
                 g["order"][2])

        for s, g in enumerate(geo):
            ch = g["ch"]
            rdmas[(s, 2)].wait()
            off = g["b1"] * 2 * ch + g["b0"] * ch
            acc_ref[pl.ds(g["own"], ch), :] = (
                x_ref[pl.ds(g["own"], ch), :]
                + rs_bufs[s][0][pl.ds(off, ch), :].astype(jnp.float32)
            ).astype(jnp.bfloat16)

        for s, g in enumerate(geo):
            ch = g["ch"]
            rdmas[(s, 4)].wait()
            off = g["b0"] * ch
            a = acc_ref[pl.ds(g["own"], ch), :].astype(jnp.float32)
            b = rs_bufs[s][1][pl.ds(off, ch), :].astype(jnp.float32)
            acc_ref[pl.ds(g["own"], ch), :] = (a + b).astype(jnp.bfloat16)

        for s, g in enumerate(geo):
            ch = g["ch"]
            rdmas[(s, 5)].wait()
            sv = (
                acc_ref[pl.ds(g["own"], ch), :].astype(jnp.float32)
                + rs_bufs[s][2][...].astype(jnp.float32)
            )
            rv = jnp.maximum(sv, 0.0)
            fv = jnp.tanh(sv) * sv * sv + rv * rv * rv
            out_ref[pl.ds(g["own"], ch), :] = fv.astype(jnp.bfloat16)
            src = chunk_src(s, g, g["c_own"])
            xchg(s, 6, src, src, g["order"][2])
            xchg(s, 7, src, src, g["order"][1])
            xchg(s, 8, src, src, g["order"][0])

        for s, g in enumerate(geo):
            rdmas[(s, 6)].wait()
            src = chunk_src(s, g, g["c_own"] ^ 1)
            xchg(s, 9, src, src, g["order"][1])
            xchg(s, 10, src, src, g["order"][0])

        for s, g in enumerate(geo):
            rdmas[(s, 7)].wait()
            src = chunk_src(s, g, g["c_own"] ^ 2)
            xchg(s, 11, src, src, g["order"][0])

        for s, g in enumerate(geo):
            rdmas[(s, 9)].wait()
            src = chunk_src(s, g, g["c_own"] ^ 3)
            xchg(s, 12, src, src, g["order"][0])

        for s in range(len(geo)):
            for k in (8, 10, 11, 12):
                rdmas[(s, k)].wait()

    scratch = [pltpu.VMEM((m, n), jnp.bfloat16)]
    for _, rows, _ in STREAMS:
        ch = rows // 8
        scratch += [
            pltpu.VMEM((4 * ch, n), jnp.bfloat16),
            pltpu.VMEM((2 * ch, n), jnp.bfloat16),
            pltpu.VMEM((ch, n), jnp.bfloat16),
        ]
    n_x = X_PER_STREAM * len(STREAMS)
    scratch += [pltpu.SemaphoreType.DMA((n_x,)), pltpu.SemaphoreType.DMA((n_x,))]

    return pl.pallas_call(
        body,
        out_shape=jax.ShapeDtypeStruct((m, n), jnp.bfloat16),
        in_specs=[pl.BlockSpec(memory_space=pltpu.VMEM)],
        out_specs=pl.BlockSpec(memory_space=pltpu.VMEM),
        scratch_shapes=scratch,
        compiler_params=pltpu.CompilerParams(collective_id=0),
    )(t)


# device time: 67456 ns/iter; 2.7063x vs baseline; 1.0337x over previous
import jax
import jax.numpy as jnp
from jax import lax
from jax.experimental import pallas as pl
from jax.experimental.pallas import tpu as pltpu

N_DEV = 8

STREAMS = (
    (0, 1408, ("z", "y", "x")),
    (1408, 1408, ("y", "x", "z")),
    (2816, 1280, ("x", "z", "y")),
)
X_PER_STREAM = 13


def kernel(t):
    m, n = t.shape

    def body(x_ref, out_ref, *rest):
        acc_ref, xv_ref = rest[0], rest[1]
        rs_bufs = [rest[2 + 3 * s : 5 + 3 * s] for s in range(len(STREAMS))]
        send_sems, recv_sems, copy_sems = rest[-3], rest[-2], rest[-1]

        my = lax.axis_index("i")
        partner = {"x": my ^ 1, "y": my ^ 3, "z": my ^ 4}
        bit = {"x": (my ^ (my >> 1)) & 1, "y": (my >> 1) & 1, "z": my >> 2}

        bsem = pltpu.get_barrier_semaphore()
        for ax in ("x", "y", "z"):
            pl.semaphore_signal(
                bsem,
                inc=1,
                device_id=(partner[ax],),
                device_id_type=pl.DeviceIdType.MESH,
            )
        pl.semaphore_wait(bsem, 3)

        geo = []
        for base, rows, order in STREAMS:
            ch = rows // 8
            b2, b1, b0 = bit[order[0]], bit[order[1]], bit[order[2]]
            g = dict(base=base, ch=ch, order=order, b2=b2, b1=b1, b0=b0)
            g["c_own"] = b2 * 4 + b1 * 2 + b0
            g["keep4"] = base + b2 * 4 * ch
            g["send4"] = base + (b2 ^ 1) * 4 * ch
            g["keep2"] = g["keep4"] + b1 * 2 * ch
            g["send2"] = g["keep4"] + (b1 ^ 1) * 2 * ch
            g["own"] = base + g["c_own"] * ch
            g["send1"] = g["keep2"] + (b0 ^ 1) * ch
            geo.append(g)

        rdmas = {}

        def xchg(s, k, src, dst, ax):
            idx = s * X_PER_STREAM + k
            r = pltpu.make_async_remote_copy(
                src_ref=src,
                dst_ref=dst,
                send_sem=send_sems.at[idx],
                recv_sem=recv_sems.at[idx],
                device_id=(partner[ax],),
                device_id_type=pl.DeviceIdType.MESH,
            )
            r.start()
            rdmas[(s, k)] = r

        def chunk_src(s, g, c):
            return out_ref.at[pl.ds(g["base"] + c * g["ch"], g["ch"])]

        cps = []
        for s, g in enumerate(geo):
            rows_ = pl.ds(g["send4"], 4 * g["ch"])
            c = pltpu.make_async_copy(x_ref.at[rows_], xv_ref.at[rows_],
                                      copy_sems.at[s])
            c.start()
            cps.append(c)
        cpk = []
        for s, g in enumerate(geo):
            rows_ = pl.ds(g["keep4"], 4 * g["ch"])
            c = pltpu.make_async_copy(x_ref.at[rows_], xv_ref.at[rows_],
                                      copy_sems.at[len(geo) + s])
            c.start()
            cpk.append(c)

        for s, g in enumerate(geo):
            ch, off = g["ch"], (g["b1"] ^ 1) * 2 * g["ch"]
            cps[s].wait()
            rows_ = pl.ds(g["send4"] + off, 2 * ch)
            acc_ref[rows_, :] = xv_ref[rows_, :].astype(jnp.bfloat16)
            xchg(s, 0, acc_ref.at[rows_], rs_bufs[s][0].at[pl.ds(off, 2 * ch)],
                 g["order"][0])

        for s, g in enumerate(geo):
            ch, off = g["ch"], g["b1"] * 2 * g["ch"]
            rows_ = pl.ds(g["send4"] + off, 2 * ch)
            acc_ref[rows_, :] = xv_ref[rows_, :].astype(jnp.bfloat16)
            for k, cb in ((1, g["b0"] ^ 1), (2, g["b0"])):
                po = off + cb * ch
                xchg(s, k, acc_ref.at[pl.ds(g["send4"] + po, ch)],
                     rs_bufs[s][0].at[pl.ds(po, ch)], g["order"][0])

        for s, g in enumerate(geo):
            ch = g["ch"]
            rdmas[(s, 0)].wait()
            cpk[s].wait()
            off = (g["b1"] ^ 1) * 2 * ch
            acc_ref[pl.ds(g["send2"], 2 * ch), :] = (
                xv_ref[pl.ds(g["send2"], 2 * ch), :]
                + rs_bufs[s][0][pl.ds(off, 2 * ch), :].astype(jnp.float32)
            ).astype(jnp.bfloat16)
            for k, cb in ((3, g["b0"] ^ 1), (4, g["b0"])):
                rows_ = pl.ds(g["send2"] + cb * ch, ch)
                xchg(s, k, acc_ref.at[rows_],
                     rs_bufs[s][1].at[pl.ds(cb * ch, ch)], g["order"][1])

        for s, g in enumerate(geo):
            ch = g["ch"]
            rdmas[(s, 1)].wait()
            off = g["b1"] * 2 * ch + (g["b0"] ^ 1) * ch
            acc_ref[pl.ds(g["send1"], ch), :] = (
                xv_ref[pl.ds(g["send1"], ch), :]
                + rs_bufs[s][0][pl.ds(off, ch), :].astype(jnp.float32)
            ).astype(jnp.bfloat16)

        for s, g in enumerate(geo):
            ch = g["ch"]
            rdmas[(s, 3)].wait()
            off = (g["b0"] ^ 1) * ch
            a = acc_ref[pl.ds(g["send1"], ch), :].astype(jnp.float32)
            b = rs_bufs[s][1][pl.ds(off, ch), :].astype(jnp.float32)
            acc_ref[pl.ds(g["send1"], ch), :] = (a + b).astype(jnp.bfloat16)
            xchg(s, 5, acc_ref.at[pl.ds(g["send1"], ch)], rs_bufs[s][2],
                 g["order"][2])

        for s, g in enumerate(geo):
            ch = g["ch"]
            rdmas[(s, 2)].wait()
            off = g["b1"] * 2 * ch + g["b0"] * ch
            acc_ref[pl.ds(g["own"], ch), :] = (
                xv_ref[pl.ds(g["own"], ch), :]
                + rs_bufs[s][0][pl.ds(off, ch), :].astype(jnp.float32)
            ).astype(jnp.bfloat16)

        for s, g in enumerate(geo):
            ch = g["ch"]
            rdmas[(s, 4)].wait()
            off = g["b0"] * ch
            a = acc_ref[pl.ds(g["own"], ch), :].astype(jnp.float32)
            b = rs_bufs[s][1][pl.ds(off, ch), :].astype(jnp.float32)
            acc_ref[pl.ds(g["own"], ch), :] = (a + b).astype(jnp.bfloat16)

        for s, g in enumerate(geo):
            ch = g["ch"]
            rdmas[(s, 5)].wait()
            sv = (
                acc_ref[pl.ds(g["own"], ch), :].astype(jnp.float32)
                + rs_bufs[s][2][...].astype(jnp.float32)
            )
            rv = jnp.maximum(sv, 0.0)
            fv = jnp.tanh(sv) * sv * sv + rv * rv * rv
            out_ref[pl.ds(g["own"], ch), :] = fv.astype(jnp.bfloat16)
            src = chunk_src(s, g, g["c_own"])
            xchg(s, 6, src, src, g["order"][2])
            xchg(s, 7, src, src, g["order"][1])
            xchg(s, 8, src, src, g["order"][0])

        for s, g in enumerate(geo):
            rdmas[(s, 6)].wait()
            src = chunk_src(s, g, g["c_own"] ^ 1)
            xchg(s, 9, src, src, g["order"][1])
            xchg(s, 10, src, src, g["order"][0])

        for s, g in enumerate(geo):
            rdmas[(s, 7)].wait()
            src = chunk_src(s, g, g["c_own"] ^ 2)
            xchg(s, 11, src, src, g["order"][0])

        for s, g in enumerate(geo):
            rdmas[(s, 9)].wait()
            src = chunk_src(s, g, g["c_own"] ^ 3)
            xchg(s, 12, src, src, g["order"][0])

        for s in range(len(geo)):
            for k in (8, 10, 11, 12):
                rdmas[(s, k)].wait()

    scratch = [
        pltpu.VMEM((m, n), jnp.bfloat16),
        pltpu.VMEM((m, n), jnp.float32),
    ]
    for _, rows, _ in STREAMS:
        ch = rows // 8
        scratch += [
            pltpu.VMEM((4 * ch, n), jnp.bfloat16),
            pltpu.VMEM((2 * ch, n), jnp.bfloat16),
            pltpu.VMEM((ch, n), jnp.bfloat16),
        ]
    n_x = X_PER_STREAM * len(STREAMS)
    scratch += [
        pltpu.SemaphoreType.DMA((n_x,)),
        pltpu.SemaphoreType.DMA((n_x,)),
        pltpu.SemaphoreType.DMA((2 * len(STREAMS),)),
    ]

    return pl.pallas_call(
        body,
        out_shape=jax.ShapeDtypeStruct((m, n), jnp.bfloat16),
        in_specs=[pl.BlockSpec(memory_space=pltpu.MemorySpace.HBM)],
        out_specs=pl.BlockSpec(memory_space=pltpu.VMEM),
        scratch_shapes=scratch,
        compiler_params=pltpu.CompilerParams(collective_id=0),
    )(t)


# device time: 67401 ns/iter; 2.7085x vs baseline; 1.0008x over previous
import jax
import jax.numpy as jnp
from jax import lax
from jax.experimental import pallas as pl
from jax.experimental.pallas import tpu as pltpu

N_DEV = 8

STREAMS = (
    (0, 1408, ("z", "y", "x")),
    (1408, 1408, ("y", "x", "z")),
    (2816, 1280, ("x", "z", "y")),
)
X_PER_STREAM = 13


def kernel(t):
    m, n = t.shape

    def body(x_ref, out_ref, *rest):
        acc_ref, xv_ref = rest[0], rest[1]
        fout_ref = acc_ref
        rs_bufs = [rest[2 + 3 * s : 5 + 3 * s] for s in range(len(STREAMS))]
        send_sems, recv_sems, copy_sems, out_sems = rest[-4:]

        my = lax.axis_index("i")
        partner = {"x": my ^ 1, "y": my ^ 3, "z": my ^ 4}
        bit = {"x": (my ^ (my >> 1)) & 1, "y": (my >> 1) & 1, "z": my >> 2}

        bsem = pltpu.get_barrier_semaphore()
        for ax in ("x", "y", "z"):
            pl.semaphore_signal(
                bsem,
                inc=1,
                device_id=(partner[ax],),
                device_id_type=pl.DeviceIdType.MESH,
            )
        pl.semaphore_wait(bsem, 3)

        geo = []
        for base, rows, order in STREAMS:
            ch = rows // 8
            b2, b1, b0 = bit[order[0]], bit[order[1]], bit[order[2]]
            g = dict(base=base, ch=ch, order=order, b2=b2, b1=b1, b0=b0)
            g["c_own"] = b2 * 4 + b1 * 2 + b0
            g["keep4"] = base + b2 * 4 * ch
            g["send4"] = base + (b2 ^ 1) * 4 * ch
            g["keep2"] = g["keep4"] + b1 * 2 * ch
            g["send2"] = g["keep4"] + (b1 ^ 1) * 2 * ch
            g["own"] = base + g["c_own"] * ch
            g["send1"] = g["keep2"] + (b0 ^ 1) * ch
            geo.append(g)

        rdmas = {}

        def xchg(s, k, src, dst, ax):
            idx = s * X_PER_STREAM + k
            r = pltpu.make_async_remote_copy(
                src_ref=src,
                dst_ref=dst,
                send_sem=send_sems.at[idx],
                recv_sem=recv_sems.at[idx],
                device_id=(partner[ax],),
                device_id_type=pl.DeviceIdType.MESH,
            )
            r.start()
            rdmas[(s, k)] = r

        def chunk_src(s, g, c):
            return fout_ref.at[pl.ds(g["base"] + c * g["ch"], g["ch"])]

        out_copies = []

        def out_chunk(s, g, c):
            rows_ = pl.ds(g["base"] + c * g["ch"], g["ch"])
            cp = pltpu.make_async_copy(fout_ref.at[rows_], out_ref.at[rows_],
                                       out_sems.at[len(out_copies)])
            cp.start()
            out_copies.append(cp)

        cpf, cpo, cpk = [], [], []
        for s, g in enumerate(geo):
            rows_ = pl.ds(g["send4"] + (g["b1"] ^ 1) * 2 * g["ch"], 2 * g["ch"])
            c = pltpu.make_async_copy(x_ref.at[rows_], xv_ref.at[rows_],
                                      copy_sems.at[s])
            c.start()
            cpf.append(c)
        for s, g in enumerate(geo):
            rows_ = pl.ds(g["send4"] + g["b1"] * 2 * g["ch"], 2 * g["ch"])
            c = pltpu.make_async_copy(x_ref.at[rows_], xv_ref.at[rows_],
                                      copy_sems.at[len(geo) + s])
            c.start()
            cpo.append(c)
        for s, g in enumerate(geo):
            rows_ = pl.ds(g["keep4"], 4 * g["ch"])
            c = pltpu.make_async_copy(x_ref.at[rows_], xv_ref.at[rows_],
                                      copy_sems.at[2 * len(geo) + s])
            c.start()
            cpk.append(c)

        for s, g in enumerate(geo):
            ch, off = g["ch"], (g["b1"] ^ 1) * 2 * g["ch"]
            cpf[s].wait()
            rows_ = pl.ds(g["send4"] + off, 2 * ch)
            acc_ref[rows_, :] = xv_ref[rows_, :].astype(jnp.bfloat16)
            xchg(s, 0, acc_ref.at[rows_], rs_bufs[s][0].at[pl.ds(off, 2 * ch)],
                 g["order"][0])

        for s, g in enumerate(geo):
            ch, off = g["ch"], g["b1"] * 2 * g["ch"]
            cpo[s].wait()
            rows_ = pl.ds(g["send4"] + off, 2 * ch)
            acc_ref[rows_, :] = xv_ref[rows_, :].astype(jnp.bfloat16)
            for k, cb in ((1, g["b0"] ^ 1), (2, g["b0"])):
                po = off + cb * ch
                xchg(s, k, acc_ref.at[pl.ds(g["send4"] + po, ch)],
                     rs_bufs[s][0].at[pl.ds(po, ch)], g["order"][0])

        for s, g in enumerate(geo):
            ch = g["ch"]
            rdmas[(s, 0)].wait()
            cpk[s].wait()
            off = (g["b1"] ^ 1) * 2 * ch
            acc_ref[pl.ds(g["send2"], 2 * ch), :] = (
                xv_ref[pl.ds(g["send2"], 2 * ch), :]
                + rs_bufs[s][0][pl.ds(off, 2 * ch), :].astype(jnp.float32)
            ).astype(jnp.bfloat16)
            for k, cb in ((3, g["b0"] ^ 1), (4, g["b0"])):
                rows_ = pl.ds(g["send2"] + cb * ch, ch)
                xchg(s, k, acc_ref.at[rows_],
                     rs_bufs[s][1].at[pl.ds(cb * ch, ch)], g["order"][1])

        for s, g in enumerate(geo):
            ch = g["ch"]
            rdmas[(s, 1)].wait()
            off = g["b1"] * 2 * ch + (g["b0"] ^ 1) * ch
            acc_ref[pl.ds(g["send1"], ch), :] = (
                xv_ref[pl.ds(g["send1"], ch), :]
                + rs_bufs[s][0][pl.ds(off, ch), :].astype(jnp.float32)
            ).astype(jnp.bfloat16)

        for s, g in enumerate(geo):
            ch = g["ch"]
            rdmas[(s, 3)].wait()
            off = (g["b0"] ^ 1) * ch
            a = acc_ref[pl.ds(g["send1"], ch), :].astype(jnp.float32)
            b = rs_bufs[s][1][pl.ds(off, ch), :].astype(jnp.float32)
            acc_ref[pl.ds(g["send1"], ch), :] = (a + b).astype(jnp.bfloat16)
            xchg(s, 5, acc_ref.at[pl.ds(g["send1"], ch)], rs_bufs[s][2],
                 g["order"][2])

        for s, g in enumerate(geo):
            ch = g["ch"]
            rdmas[(s, 2)].wait()
            off = g["b1"] * 2 * ch + g["b0"] * ch
            acc_ref[pl.ds(g["own"], ch), :] = (
                xv_ref[pl.ds(g["own"], ch), :]
                + rs_bufs[s][0][pl.ds(off, ch), :].astype(jnp.float32)
            ).astype(jnp.bfloat16)

        for s, g in enumerate(geo):
            ch = g["ch"]
            rdmas[(s, 4)].wait()
            off = g["b0"] * ch
            a = acc_ref[pl.ds(g["own"], ch), :].astype(jnp.float32)
            b = rs_bufs[s][1][pl.ds(off, ch), :].astype(jnp.float32)
            acc_ref[pl.ds(g["own"], ch), :] = (a + b).astype(jnp.bfloat16)

        for s, g in enumerate(geo):
            ch = g["ch"]
            rdmas[(s, 5)].wait()
            sv = (
                acc_ref[pl.ds(g["own"], ch), :].astype(jnp.float32)
                + rs_bufs[s][2][...].astype(jnp.float32)
            )
            rv = jnp.maximum(sv, 0.0)
            fv = jnp.tanh(sv) * sv * sv + rv * rv * rv
            fout_ref[pl.ds(g["own"], ch), :] = fv.astype(jnp.bfloat16)
            src = chunk_src(s, g, g["c_own"])
            xchg(s, 6, src, src, g["order"][2])
            xchg(s, 7, src, src, g["order"][1])
            xchg(s, 8, src, src, g["order"][0])
            out_chunk(s, g, g["c_own"])

        for s, g in enumerate(geo):
            rdmas[(s, 6)].wait()
            src = chunk_src(s, g, g["c_own"] ^ 1)
            xchg(s, 9, src, src, g["order"][1])
            xchg(s, 10, src, src, g["order"][0])
            out_chunk(s, g, g["c_own"] ^ 1)

        for s, g in enumerate(geo):
            rdmas[(s, 7)].wait()
            src = chunk_src(s, g, g["c_own"] ^ 2)
            xchg(s, 11, src, src, g["order"][0])
            out_chunk(s, g, g["c_own"] ^ 2)

        for s, g in enumerate(geo):
            rdmas[(s, 9)].wait()
            src = chunk_src(s, g, g["c_own"] ^ 3)
            xchg(s, 12, src, src, g["order"][0])
            out_chunk(s, g, g["c_own"] ^ 3)

        for s, g in enumerate(geo):
            for k, cbit in ((8, 4), (10, 5), (11, 6), (12, 7)):
                rdmas[(s, k)].wait()
                out_chunk(s, g, g["c_own"] ^ cbit)

        for cp in out_copies:
            cp.wait()

    scratch = [
        pltpu.VMEM((m, n), jnp.bfloat16),
        pltpu.VMEM((m, n), jnp.float32),
    ]
    for _, rows, _ in STREAMS:
        ch = rows // 8
        scratch += [
            pltpu.VMEM((4 * ch, n), jnp.bfloat16),
            pltpu.VMEM((2 * ch, n), jnp.bfloat16),
            pltpu.VMEM((ch, n), jnp.bfloat16),
        ]
    n_x = X_PER_STREAM * len(STREAMS)
    scratch += [
        pltpu.SemaphoreType.DMA((n_x,)),
        pltpu.SemaphoreType.DMA((n_x,)),
        pltpu.SemaphoreType.DMA((3 * len(STREAMS),)),
        pltpu.SemaphoreType.DMA((8 * len(STREAMS),)),
    ]

    return pl.pallas_call(
        body,
        out_shape=jax.ShapeDtypeStruct((m, n), jnp.bfloat16),
        in_specs=[pl.BlockSpec(memory_space=pltpu.MemorySpace.HBM)],
        out_specs=pl.BlockSpec(memory_space=pltpu.MemorySpace.HBM),
        scratch_shapes=scratch,
        compiler_params=pltpu.CompilerParams(collective_id=0),
    )(t)
